# baseline (speedup 1.0000x reference)
import functools

import jax
import jax.numpy as jnp
from jax.experimental import pallas as pl
from jax.experimental.pallas import tpu as pltpu


def _gat_kernel(x_ref, w_ref, as_ref, ad_ref, mask_ref, bias_ref, out_ref,
                hta_scr, asrc_scr, adstt_scr, *, n, t_steps, heads, dim, bj):
    j = pl.program_id(0)

    @pl.when(j == 0)
    def _prep():
        w = w_ref[...]
        ones_col = jnp.ones((n, 1), dtype=jnp.bfloat16)
        for t in range(t_steps):
            ht = jnp.dot(x_ref[:, t, :], w,
                         preferred_element_type=jnp.float32)
            asrc_scr[t] = jnp.dot(ht, as_ref[...],
                                  preferred_element_type=jnp.float32)
            adstt_scr[t] = jax.lax.dot_general(
                ad_ref[...], ht, (((1,), (1,)), ((), ())),
                preferred_element_type=jnp.float32)
            ht_bf = ht.astype(jnp.bfloat16)
            pieces = []
            for hh in range(heads):
                pieces.append(ht_bf[:, hh * dim:(hh + 1) * dim])
                pieces.append(ones_col)
            hta_scr[t] = jnp.concatenate(pieces, axis=1)

    col0 = j * bj
    m = mask_ref[...]
    rows = jax.lax.broadcasted_iota(jnp.int32, (n, bj), 0)
    cols = jax.lax.broadcasted_iota(jnp.int32, (n, bj), 1) + col0
    maskadd = jnp.where((m != 0) | (rows == cols), 0.0, -1e30)
    b = bias_ref[...]
    inv_h = jnp.float32(1.0 / heads)
    da = dim + 1
    for t in range(t_steps):
        acc = None
        for hh in range(heads):
            s = (asrc_scr[t, :, hh:hh + 1]
                 + adstt_scr[t, hh:hh + 1, pl.ds(col0, bj)]
                 + maskadd)
            s = jnp.maximum(s, 0.2 * s)
            ex = jnp.exp2(s).astype(jnp.bfloat16)
            o_aug = jax.lax.dot_general(
                hta_scr[t, :, hh * da:(hh + 1) * da], ex,
                (((0,), (0,)), ((), ())),
                preferred_element_type=jnp.float32)
            o = o_aug[:dim, :] / (o_aug[dim:, :] + 1e-16)
            acc = o if acc is None else acc + o
        out_ref[:, t * dim:(t + 1) * dim] = jnp.transpose(acc * inv_h) + b


def kernel(x, node_matrix, W, att_src, att_dst, bias):
    n, t_steps, in_dim = x.shape
    heads, dim = att_src.shape[1], att_src.shape[2]
    hc = heads * dim
    bj = 256
    nj = n // bj

    eye = jnp.eye(heads, dtype=jnp.float32)
    log2e = jnp.float32(1.4426950408889634)
    as_bd = (att_src.reshape(heads, dim)[:, :, None]
             * eye[:, None, :]).reshape(hc, heads) * log2e
    ad_bd = (att_dst.reshape(heads, dim)[:, None, :]
             * eye[:, :, None]).reshape(heads, hc) * log2e
    bias_row = bias.reshape(1, dim).astype(jnp.float32)

    body = functools.partial(_gat_kernel, n=n, t_steps=t_steps,
                             heads=heads, dim=dim, bj=bj)
    out = pl.pallas_call(
        body,
        grid=(nj,),
        in_specs=[
            pl.BlockSpec((n, t_steps, in_dim), lambda j: (0, 0, 0)),
            pl.BlockSpec((in_dim, hc), lambda j: (0, 0)),
            pl.BlockSpec((hc, heads), lambda j: (0, 0)),
            pl.BlockSpec((heads, hc), lambda j: (0, 0)),
            pl.BlockSpec((n, bj), lambda j: (0, j)),
            pl.BlockSpec((1, dim), lambda j: (0, 0)),
        ],
        out_specs=pl.BlockSpec((bj, t_steps * dim), lambda j: (j, 0)),
        out_shape=jax.ShapeDtypeStruct((n, t_steps * dim), jnp.float32),
        scratch_shapes=[
            pltpu.VMEM((t_steps, n, heads * (dim + 1)), jnp.bfloat16),
            pltpu.VMEM((t_steps, n, heads), jnp.float32),
            pltpu.VMEM((t_steps, heads, n), jnp.float32),
        ],
        compiler_params=pltpu.CompilerParams(
            dimension_semantics=("arbitrary",)),
    )(x.astype(jnp.float32), W, as_bd, ad_bd, node_matrix, bias_row)
    return out.reshape(n, t_steps, dim)


# speedup vs baseline: 1.6471x; 1.6471x over previous
import functools

import jax
import jax.numpy as jnp
from jax.experimental import pallas as pl
from jax.experimental.pallas import tpu as pltpu


def _gat_kernel(x_ref, w_ref, as_ref, ad_ref, mask_ref, bias_ref, out_ref,
                *, n, t_steps, heads, dim):
    m = mask_ref[...]
    rows = jax.lax.broadcasted_iota(jnp.int32, (n, n), 0)
    cols = jax.lax.broadcasted_iota(jnp.int32, (n, n), 1)
    maskadd = jnp.where((m != 0) | (rows == cols), 0.0,
                        -1e30).astype(jnp.bfloat16)
    w = w_ref[...]
    b = bias_ref[...]
    inv_h = jnp.float32(1.0 / heads)
    ones_col = jnp.ones((n, 1), dtype=jnp.bfloat16)
    slope = jnp.bfloat16(0.2)
    for t in range(t_steps):
        ht = jnp.dot(x_ref[:, t, :], w,
                     preferred_element_type=jnp.float32)
        a_src = jnp.dot(ht, as_ref[...],
                        preferred_element_type=jnp.float32
                        ).astype(jnp.bfloat16)
        a_dst = jax.lax.dot_general(
            ad_ref[...], ht, (((1,), (1,)), ((), ())),
            preferred_element_type=jnp.float32
            ).astype(jnp.bfloat16)
        ht_bf = ht.astype(jnp.bfloat16)
        acc = None
        for hh in range(heads):
            s = a_src[:, hh:hh + 1] + a_dst[hh:hh + 1, :] + maskadd
            s = jnp.maximum(s, slope * s)
            ex = jnp.exp2(s)
            lhs = jnp.concatenate(
                [ht_bf[:, hh * dim:(hh + 1) * dim], ones_col], axis=1)
            o_aug = jax.lax.dot_general(
                lhs, ex, (((0,), (0,)), ((), ())),
                preferred_element_type=jnp.float32)
            o = o_aug[:dim, :] / (o_aug[dim:, :] + 1e-16)
            acc = o if acc is None else acc + o
        out_ref[t * dim:(t + 1) * dim, :] = acc * inv_h + b


def kernel(x, node_matrix, W, att_src, att_dst, bias):
    n, t_steps, in_dim = x.shape
    heads, dim = att_src.shape[1], att_src.shape[2]
    hc = heads * dim

    eye = jnp.eye(heads, dtype=jnp.float32)
    log2e = jnp.float32(1.4426950408889634)
    as_bd = (att_src.reshape(heads, dim)[:, :, None]
             * eye[:, None, :]).reshape(hc, heads) * log2e
    ad_bd = (att_dst.reshape(heads, dim)[:, None, :]
             * eye[:, :, None]).reshape(heads, hc) * log2e
    bias_col = bias.reshape(dim, 1).astype(jnp.float32)

    body = functools.partial(_gat_kernel, n=n, t_steps=t_steps,
                             heads=heads, dim=dim)
    out_t = pl.pallas_call(
        body,
        out_shape=jax.ShapeDtypeStruct((t_steps * dim, n), jnp.float32),
    )(x.astype(jnp.float32), W, as_bd, ad_bd, node_matrix, bias_col)
    return jnp.transpose(out_t.reshape(t_steps, dim, n), (2, 0, 1))
